# baseline (speedup 1.0000x reference)
import functools

import jax
import jax.numpy as jnp
from jax import lax
from jax.experimental import pallas as pl
from jax.experimental.pallas import tpu as pltpu
from jax.experimental.pallas import tpu_sc as plsc

ROWS = 4 * 8192
CHANNELS = 2048
MID0, MID = 512, 1024
NWORKERS = 32
ROWS_PER_W = ROWS // NWORKERS
CHUNK = 8

_mesh = plsc.VectorSubcoreMesh(core_axis_name="c", subcore_axis_name="s")


@functools.partial(
    pl.kernel,
    mesh=_mesh,
    out_type=jax.ShapeDtypeStruct((ROWS, CHANNELS), jnp.float32),
    compiler_params=pltpu.CompilerParams(needs_layout_passes=False),
    scratch_types=[
        pltpu.VMEM((CHUNK, MID), jnp.float32),
        pltpu.VMEM((CHUNK, MID), jnp.float32),
    ],
)
def _permute(x_hbm, o_hbm, in_v, out_v):
    wid = lax.axis_index("s") * 2 + lax.axis_index("c")
    base = wid * ROWS_PER_W

    pltpu.sync_copy(
        x_hbm.at[pl.ds(base, ROWS_PER_W), pl.ds(0, MID0)],
        o_hbm.at[pl.ds(base, ROWS_PER_W), pl.ds(0, MID0)],
    )
    pltpu.sync_copy(
        x_hbm.at[pl.ds(base, ROWS_PER_W), pl.ds(MID0 + MID, MID0)],
        o_hbm.at[pl.ds(base, ROWS_PER_W), pl.ds(MID0 + MID, MID0)],
    )

    lane = lax.iota(jnp.int32, 16)
    pattern = (lane >> 1) + ((lane & 1) << 9)

    def chunk_body(g, carry):
        r0 = base + g * CHUNK
        pltpu.sync_copy(x_hbm.at[pl.ds(r0, CHUNK), pl.ds(MID0, MID)], in_v)
        zero = lane & 0
        for r in range(CHUNK):
            rowv = zero + r
            for k in range(MID // 16):
                idxv = pattern + (8 * k)
                out_v[r, pl.ds(16 * k, 16)] = plsc.load_gather(
                    in_v, [rowv, idxv])
        pltpu.sync_copy(out_v, o_hbm.at[pl.ds(r0, CHUNK), pl.ds(MID0, MID)])
        return carry

    lax.fori_loop(0, ROWS_PER_W // CHUNK, chunk_body, 0)


def kernel(inputs):
    x = inputs.reshape(ROWS, CHANNELS)
    return _permute(x).reshape(inputs.shape)
